# baseline (speedup 1.0000x reference)
import functools

import jax
import jax.numpy as jnp
from jax import lax
from jax.experimental import pallas as pl
from jax.experimental.pallas import tpu as pltpu
from jax.experimental.pallas import tpu_sc as plsc

_B = 1024
_HIST = 50
_NEG = 10
_D = 64
_RPE = 64
_ROWS = _B * _RPE
_CHUNK = 128
_NCHUNKS = _ROWS // _CHUNK
_NW = 32
_TILES = 500000
_DROWS = 7813


def _sc_gather(tidx2d, sidx, table2, dtab128):
    mesh = plsc.VectorSubcoreMesh(core_axis_name="c", subcore_axis_name="s")

    @functools.partial(
        pl.kernel,
        mesh=mesh,
        out_type=(
            jax.ShapeDtypeStruct((_ROWS, 128), jnp.float32),
            jax.ShapeDtypeStruct((_B, 128), jnp.float32),
        ),
        scratch_types=(
            pltpu.VMEM((_CHUNK,), jnp.int32),
            pltpu.VMEM((_CHUNK, 128), jnp.float32),
            pltpu.VMEM((_B // _NW,), jnp.int32),
            pltpu.VMEM((_B // _NW, 128), jnp.float32),
            pltpu.SemaphoreType.DMA,
        ),
    )
    def k(tidx_hbm, sidx_hbm, table_hbm, dtab_hbm, rows_out,
          drows_out, tidx_v, out_v, ridx_v, drow_v, sem):
        wid = lax.axis_index("s") * 2 + lax.axis_index("c")

        bpw = _B // _NW
        pltpu.sync_copy(sidx_hbm.at[pl.ds(wid * bpw, bpw)], ridx_v)
        pltpu.async_copy(dtab_hbm.at[ridx_v], drow_v, sem).wait()
        pltpu.sync_copy(drow_v, drows_out.at[pl.ds(wid * bpw, bpw)])

        def body(t, carry):
            c = wid + t * _NW
            pltpu.sync_copy(tidx_hbm.at[c], tidx_v)
            pltpu.async_copy(table_hbm.at[tidx_v], out_v, sem).wait()
            pltpu.sync_copy(out_v, rows_out.at[pl.ds(c * _CHUNK, _CHUNK)])
            return carry

        lax.fori_loop(0, _NCHUNKS // _NW, body, 0)

    return k(tidx2d, sidx, table2, dtab128)


def _tc_math(rows3, parity, t_times, h_times, h_time_mask, drows, dcol):
    tb = 256
    grid = _B // tb

    def body(rows_ref, par_ref, tt_ref, ht_ref, mask_ref, drows_ref, dcol_ref,
             out_ref):
        lane = lax.broadcasted_iota(jnp.int32, (tb, 128), 1)
        delta = jnp.sum(
            jnp.where(lane == dcol_ref[...], drows_ref[...], 0.0),
            axis=1, keepdims=True)
        pair = rows_ref[...]
        rows = jnp.where(par_ref[...][:, :, None] == 1,
                         pair[:, :, _D:], pair[:, :, :_D])
        s = rows[:, 0, :]
        t = rows[:, 1, :]
        h = rows[:, 2:2 + _HIST, :]
        n = rows[:, 2 + _HIST:2 + _HIST + _NEG, :]

        d_sh = jnp.sum((s[:, None, :] - h) ** 2, axis=2)
        m = jnp.max(-d_sh, axis=1, keepdims=True)
        e = jnp.exp(-d_sh - m)
        att = e / jnp.sum(e, axis=1, keepdims=True)

        dt = jnp.abs(tt_ref[...] - ht_ref[...])
        c = att * jnp.exp(delta * dt) * mask_ref[...]

        d_st = jnp.sum((s - t) ** 2, axis=1)
        d_ht = jnp.sum((h - t[:, None, :]) ** 2, axis=2)
        p_lambda = -d_st - jnp.sum(c * d_ht, axis=1)

        csum = jnp.sum(c, axis=1)
        q = jnp.sum(c * jnp.sum(h * h, axis=2), axis=1)
        hbar = jnp.sum(c[:, :, None] * h, axis=1)
        n_norm = jnp.sum(n * n, axis=2)
        hdot = jnp.sum(hbar[:, None, :] * n, axis=2)
        d_sn = jnp.sum((s[:, None, :] - n) ** 2, axis=2)
        n_lambda = -d_sn - q[:, None] - csum[:, None] * n_norm + 2.0 * hdot

        pos = -jnp.log(jax.nn.sigmoid(p_lambda) + 1e-6)
        neg = jnp.sum(jnp.log(jax.nn.sigmoid(-n_lambda) + 1e-6), axis=1)
        out_ref[...] = pos - neg

    return pl.pallas_call(
        body,
        grid=(grid,),
        in_specs=[
            pl.BlockSpec((tb, _RPE, 128), lambda i: (i, 0, 0)),
            pl.BlockSpec((tb, _RPE), lambda i: (i, 0)),
            pl.BlockSpec((tb, 1), lambda i: (i, 0)),
            pl.BlockSpec((tb, _HIST), lambda i: (i, 0)),
            pl.BlockSpec((tb, _HIST), lambda i: (i, 0)),
            pl.BlockSpec((tb, 128), lambda i: (i, 0)),
            pl.BlockSpec((tb, 1), lambda i: (i, 0)),
        ],
        out_specs=pl.BlockSpec((tb,), lambda i: (i,)),
        out_shape=jax.ShapeDtypeStruct((_B,), jnp.float32),
    )(rows3, parity, t_times, h_times, h_time_mask, drows, dcol)


def kernel(s_nodes, t_nodes, t_times, h_nodes, h_times, h_time_mask,
           n_nodes, node_emb, delta_tab):
    pad = jnp.zeros((_B, _RPE - 2 - _HIST - _NEG), jnp.int32)
    idx = jnp.concatenate([s_nodes, t_nodes, h_nodes, n_nodes, pad], axis=1)
    tidx2d = jnp.right_shift(idx, 1).reshape(_NCHUNKS, _CHUNK)
    parity = jnp.bitwise_and(idx, 1)
    s_idx = s_nodes.reshape(_B)
    table2 = node_emb.reshape(_TILES, 128)
    dtab128 = jnp.pad(delta_tab.reshape(-1),
                      (0, _DROWS * 128 - 1000000)).reshape(_DROWS, 128)
    rows, drows = _sc_gather(tidx2d, jnp.right_shift(s_idx, 7),
                             table2, dtab128)
    dcol = jnp.bitwise_and(s_idx, 127).reshape(_B, 1)
    rows3 = rows.reshape(_B, _RPE, 128)
    return _tc_math(rows3, parity, t_times, h_times, h_time_mask, drows, dcol)


# speedup vs baseline: 1.9384x; 1.9384x over previous
import functools

import jax
import jax.numpy as jnp
from jax import lax
from jax.experimental import pallas as pl
from jax.experimental.pallas import tpu as pltpu
from jax.experimental.pallas import tpu_sc as plsc

_B = 1024
_HIST = 50
_NEG = 10
_D = 64
_RPE = 64
_ROWS = _B * _RPE
_CHUNK = 128
_NCHUNKS = _ROWS // _CHUNK
_NW = 32
_TILES = 125000
_DROWS = 7813


def _sc_gather(tidx2d, sub2d, sidx, table3, dtab128):
    mesh = plsc.VectorSubcoreMesh(core_axis_name="c", subcore_axis_name="s")

    @functools.partial(
        pl.kernel,
        mesh=mesh,
        out_type=(
            jax.ShapeDtypeStruct((_ROWS, _D), jnp.float32),
            jax.ShapeDtypeStruct((_B, 128), jnp.float32),
        ),
        scratch_types=(
            pltpu.VMEM((_CHUNK,), jnp.int32),
            pltpu.VMEM((_CHUNK,), jnp.int32),
            pltpu.VMEM((_CHUNK, _D), jnp.float32),
            pltpu.VMEM((_B // _NW,), jnp.int32),
            pltpu.VMEM((_B // _NW, 128), jnp.float32),
            pltpu.SemaphoreType.DMA,
        ),
    )
    def k(tidx_hbm, sub_hbm, sidx_hbm, table_hbm, dtab_hbm, rows_out,
          drows_out, tidx_v, sub_v, out_v, ridx_v, drow_v, sem):
        wid = lax.axis_index("s") * 2 + lax.axis_index("c")

        bpw = _B // _NW
        pltpu.sync_copy(sidx_hbm.at[pl.ds(wid * bpw, bpw)], ridx_v)
        pltpu.async_copy(dtab_hbm.at[ridx_v], drow_v, sem).wait()
        pltpu.sync_copy(drow_v, drows_out.at[pl.ds(wid * bpw, bpw)])

        def body(t, carry):
            c = wid + t * _NW
            pltpu.sync_copy(tidx_hbm.at[c], tidx_v)
            pltpu.sync_copy(sub_hbm.at[c], sub_v)

            def grp(g, carry2):
                tv = tidx_v[pl.ds(g * 16, 16)]
                sv = sub_v[pl.ds(g * 16, 16)]
                for l in range(16):
                    j = g * 16 + l
                    pltpu.async_copy(table_hbm.at[tv[l], sv[l]],
                                     out_v.at[j], sem)
                return carry2

            lax.fori_loop(0, _CHUNK // 16, grp, 0)
            pltpu.make_async_copy(
                rows_out.at[pl.ds(0, _CHUNK)], out_v, sem).wait()
            pltpu.sync_copy(out_v, rows_out.at[pl.ds(c * _CHUNK, _CHUNK)])
            return carry

        lax.fori_loop(0, _NCHUNKS // _NW, body, 0)

    return k(tidx2d, sub2d, sidx, table3, dtab128)


def _tc_math(rows3, t_times, h_times, h_time_mask, drows, dcol):
    tb = 256
    grid = _B // tb

    def body(rows_ref, tt_ref, ht_ref, mask_ref, drows_ref, dcol_ref, out_ref):
        lane = lax.broadcasted_iota(jnp.int32, (tb, 128), 1)
        delta = jnp.sum(
            jnp.where(lane == dcol_ref[...], drows_ref[...], 0.0),
            axis=1, keepdims=True)
        rows = rows_ref[...]
        s = rows[:, 0, :]
        t = rows[:, 1, :]
        h = rows[:, 2:2 + _HIST, :]
        n = rows[:, 2 + _HIST:2 + _HIST + _NEG, :]

        d_sh = jnp.sum((s[:, None, :] - h) ** 2, axis=2)
        m = jnp.max(-d_sh, axis=1, keepdims=True)
        e = jnp.exp(-d_sh - m)
        att = e / jnp.sum(e, axis=1, keepdims=True)

        dt = jnp.abs(tt_ref[...] - ht_ref[...])
        c = att * jnp.exp(delta * dt) * mask_ref[...]

        d_st = jnp.sum((s - t) ** 2, axis=1)
        d_ht = jnp.sum((h - t[:, None, :]) ** 2, axis=2)
        p_lambda = -d_st - jnp.sum(c * d_ht, axis=1)

        csum = jnp.sum(c, axis=1)
        q = jnp.sum(c * jnp.sum(h * h, axis=2), axis=1)
        hbar = jnp.sum(c[:, :, None] * h, axis=1)
        n_norm = jnp.sum(n * n, axis=2)
        hdot = jnp.sum(hbar[:, None, :] * n, axis=2)
        d_sn = jnp.sum((s[:, None, :] - n) ** 2, axis=2)
        n_lambda = -d_sn - q[:, None] - csum[:, None] * n_norm + 2.0 * hdot

        pos = -jnp.log(jax.nn.sigmoid(p_lambda) + 1e-6)
        neg = jnp.sum(jnp.log(jax.nn.sigmoid(-n_lambda) + 1e-6), axis=1)
        out_ref[...] = pos - neg

    return pl.pallas_call(
        body,
        grid=(grid,),
        in_specs=[
            pl.BlockSpec((tb, _RPE, _D), lambda i: (i, 0, 0)),
            pl.BlockSpec((tb, 1), lambda i: (i, 0)),
            pl.BlockSpec((tb, _HIST), lambda i: (i, 0)),
            pl.BlockSpec((tb, _HIST), lambda i: (i, 0)),
            pl.BlockSpec((tb, 128), lambda i: (i, 0)),
            pl.BlockSpec((tb, 1), lambda i: (i, 0)),
        ],
        out_specs=pl.BlockSpec((tb,), lambda i: (i,)),
        out_shape=jax.ShapeDtypeStruct((_B,), jnp.float32),
    )(rows3, t_times, h_times, h_time_mask, drows, dcol)


def kernel(s_nodes, t_nodes, t_times, h_nodes, h_times, h_time_mask,
           n_nodes, node_emb, delta_tab):
    pad = jnp.zeros((_B, _RPE - 2 - _HIST - _NEG), jnp.int32)
    idx = jnp.concatenate([s_nodes, t_nodes, h_nodes, n_nodes, pad], axis=1)
    tidx2d = jnp.right_shift(idx, 3).reshape(_NCHUNKS, _CHUNK)
    sub2d = jnp.bitwise_and(idx, 7).reshape(_NCHUNKS, _CHUNK)
    s_idx = s_nodes.reshape(_B)
    table3 = node_emb.reshape(_TILES, 8, _D)
    dtab128 = jnp.pad(delta_tab.reshape(-1),
                      (0, _DROWS * 128 - 1000000)).reshape(_DROWS, 128)
    rows, drows = _sc_gather(tidx2d, sub2d, jnp.right_shift(s_idx, 7),
                             table3, dtab128)
    dcol = jnp.bitwise_and(s_idx, 127).reshape(_B, 1)
    rows3 = rows.reshape(_B, _RPE, _D)
    return _tc_math(rows3, t_times, h_times, h_time_mask, drows, dcol)


# speedup vs baseline: 1.9524x; 1.0073x over previous
import functools

import jax
import jax.numpy as jnp
from jax import lax
from jax.experimental import pallas as pl
from jax.experimental.pallas import tpu as pltpu
from jax.experimental.pallas import tpu_sc as plsc

_B = 1024
_HIST = 50
_NEG = 10
_D = 64
_RPE = 64
_ROWS = _B * _RPE
_CHUNK = 128
_NCHUNKS = _ROWS // _CHUNK
_NW = 32
_TILES = 125000
_DROWS = 7813


def _sc_gather(tidx2d, sub2d, sidx, table3, dtab128):
    mesh = plsc.VectorSubcoreMesh(core_axis_name="c", subcore_axis_name="s")

    @functools.partial(
        pl.kernel,
        mesh=mesh,
        out_type=(
            jax.ShapeDtypeStruct((_ROWS, _D), jnp.float32),
            jax.ShapeDtypeStruct((_B, 128), jnp.float32),
        ),
        scratch_types=(
            pltpu.VMEM((_CHUNK,), jnp.int32),
            pltpu.VMEM((_CHUNK,), jnp.int32),
            pltpu.VMEM((_CHUNK, _D), jnp.float32),
            pltpu.VMEM((_B // _NW,), jnp.int32),
            pltpu.VMEM((_B // _NW, 128), jnp.float32),
            pltpu.SemaphoreType.DMA,
        ),
    )
    def k(tidx_hbm, sub_hbm, sidx_hbm, table_hbm, dtab_hbm, rows_out,
          drows_out, tidx_v, sub_v, out_v, ridx_v, drow_v, sem):
        wid = lax.axis_index("s") * 2 + lax.axis_index("c")

        bpw = _B // _NW
        pltpu.sync_copy(sidx_hbm.at[pl.ds(wid * bpw, bpw)], ridx_v)
        pltpu.async_copy(dtab_hbm.at[ridx_v], drow_v, sem).wait()
        pltpu.sync_copy(drow_v, drows_out.at[pl.ds(wid * bpw, bpw)])

        def body(t, carry):
            c = wid + t * _NW
            pltpu.sync_copy(tidx_hbm.at[c], tidx_v)
            pltpu.sync_copy(sub_hbm.at[c], sub_v)

            def grp(g, carry2):
                tv = tidx_v[pl.ds(g * 16, 16)]
                sv = sub_v[pl.ds(g * 16, 16)]
                for l in range(16):
                    j = g * 16 + l
                    pltpu.async_copy(table_hbm.at[tv[l], sv[l]],
                                     out_v.at[j], sem)
                return carry2

            lax.fori_loop(0, _CHUNK // 16, grp, 0)
            pltpu.make_async_copy(
                rows_out.at[pl.ds(0, _CHUNK)], out_v, sem).wait()
            pltpu.sync_copy(out_v, rows_out.at[pl.ds(c * _CHUNK, _CHUNK)])
            return carry

        lax.fori_loop(0, _NCHUNKS // _NW, body, 0)

    return k(tidx2d, sub2d, sidx, table3, dtab128)


def _tc_math(rows3, t_times, h_times, h_time_mask, drows, dcol):
    tb = 256
    grid = _B // tb

    def body(rows_ref, tt_ref, ht_ref, mask_ref, drows_ref, dcol_ref, out_ref):
        lane = lax.broadcasted_iota(jnp.int32, (tb, 128), 1)
        delta = jnp.sum(
            jnp.where(lane == dcol_ref[...], drows_ref[...], 0.0),
            axis=1, keepdims=True)
        rows = rows_ref[...]
        s = rows[:, 0, :]
        t = rows[:, 1, :]
        h = rows[:, 2:2 + _HIST, :]
        n = rows[:, 2 + _HIST:2 + _HIST + _NEG, :]

        d_sh = jnp.sum((s[:, None, :] - h) ** 2, axis=2)
        m = jnp.max(-d_sh, axis=1, keepdims=True)
        e = jnp.exp(-d_sh - m)
        att = e / jnp.sum(e, axis=1, keepdims=True)

        dt = jnp.abs(tt_ref[...] - ht_ref[...])
        c = att * jnp.exp(delta * dt) * mask_ref[...]

        d_st = jnp.sum((s - t) ** 2, axis=1)
        d_ht = jnp.sum((h - t[:, None, :]) ** 2, axis=2)
        p_lambda = -d_st - jnp.sum(c * d_ht, axis=1)

        csum = jnp.sum(c, axis=1)
        q = jnp.sum(c * jnp.sum(h * h, axis=2), axis=1)
        hbar = jnp.sum(c[:, :, None] * h, axis=1)
        n_norm = jnp.sum(n * n, axis=2)
        hdot = jnp.sum(hbar[:, None, :] * n, axis=2)
        d_sn = jnp.sum((s[:, None, :] - n) ** 2, axis=2)
        n_lambda = -d_sn - q[:, None] - csum[:, None] * n_norm + 2.0 * hdot

        pos = -jnp.log(jax.nn.sigmoid(p_lambda) + 1e-6)
        neg = jnp.sum(jnp.log(jax.nn.sigmoid(-n_lambda) + 1e-6), axis=1)
        out_ref[...] = pos - neg

    return pl.pallas_call(
        body,
        grid=(grid,),
        in_specs=[
            pl.BlockSpec((tb, _RPE, _D), lambda i: (i, 0, 0)),
            pl.BlockSpec((tb, 1), lambda i: (i, 0)),
            pl.BlockSpec((tb, _HIST), lambda i: (i, 0)),
            pl.BlockSpec((tb, _HIST), lambda i: (i, 0)),
            pl.BlockSpec((tb, 128), lambda i: (i, 0)),
            pl.BlockSpec((tb, 1), lambda i: (i, 0)),
        ],
        out_specs=pl.BlockSpec((tb,), lambda i: (i,)),
        out_shape=jax.ShapeDtypeStruct((_B,), jnp.float32),
    )(rows3, t_times, h_times, h_time_mask, drows, dcol)


def kernel(s_nodes, t_nodes, t_times, h_nodes, h_times, h_time_mask,
           n_nodes, node_emb, delta_tab):
    pad = jnp.zeros((_B, _RPE - 2 - _HIST - _NEG), jnp.int32)
    idx = jnp.concatenate([s_nodes, t_nodes, h_nodes, n_nodes, pad], axis=1)
    tidx2d = jnp.right_shift(idx, 3).reshape(_NCHUNKS, _CHUNK)
    sub2d = jnp.bitwise_and(idx, 7).reshape(_NCHUNKS, _CHUNK)
    s_idx = s_nodes.reshape(_B)
    table3 = node_emb.reshape(_TILES, 8, _D)
    dtab128 = jnp.pad(delta_tab,
                      ((0, _DROWS * 128 - 1000000), (0, 0))).reshape(_DROWS, 128)
    rows, drows = _sc_gather(tidx2d, sub2d, jnp.right_shift(s_idx, 7),
                             table3, dtab128)
    dcol = jnp.bitwise_and(s_idx, 127).reshape(_B, 1)
    rows3 = rows.reshape(_B, _RPE, _D)
    return _tc_math(rows3, t_times, h_times, h_time_mask, drows, dcol)
